# baseline (speedup 1.0000x reference)
import jax
import jax.numpy as jnp
from jax.experimental import pallas as pl
from jax.experimental.pallas import tpu as pltpu

_B = 64
_HW = 169
_NC = 20
_NA = 5
_AW = (1.3221, 3.19275, 5.05587, 9.47112, 11.2364)
_AH = (1.73145, 4.00944, 8.09892, 4.84053, 10.0071)
_LAMBDA_COORD = 5.0
_LAMBDA_NOOBJ = 0.5


_BBLK = 64
_NSTEP = _B // _BBLK


def _loss_kernel(pred_ref, tgt_ref, o0_ref, o1_ref, o2_ref, o3_ref, acc_ref):
    idx = jax.lax.broadcasted_iota(jnp.int32, (_BBLK, _HW), 1)
    gx = (idx % 13).astype(jnp.float32)
    gy = (idx // 13).astype(jnp.float32)

    gt_conf = tgt_ref[20]
    gt_x = tgt_ref[21]
    gt_y = tgt_ref[22]
    gt_w = tgt_ref[23]
    gt_h = tgt_ref[24]
    b_l = gt_x - gt_w * 0.5
    b_t = gt_y - gt_h * 0.5
    b_r = gt_x + gt_w * 0.5
    b_b = gt_y + gt_h * 0.5
    area_b = gt_w * gt_h

    best_iou = None
    for a in range(_NA):
        base = a * 25
        conf = jax.nn.sigmoid(pred_ref[base + 20])
        px = jax.nn.sigmoid(pred_ref[base + 21]) + gx
        py = jax.nn.sigmoid(pred_ref[base + 22]) + gy
        pw = jnp.exp(pred_ref[base + 23]) * _AW[a]
        ph = jnp.exp(pred_ref[base + 24]) * _AH[a]

        a_l = px - pw * 0.5
        a_t = py - ph * 0.5
        a_r = px + pw * 0.5
        a_b = py + ph * 0.5
        wi = jnp.clip(jnp.minimum(a_r, b_r) - jnp.maximum(a_l, b_l), 0.0)
        hi = jnp.clip(jnp.minimum(a_b, b_b) - jnp.maximum(a_t, b_t), 0.0)
        inter = wi * hi
        iou = inter / (pw * ph + area_b - inter + 1e-8)

        if a == 0:
            best_iou = iou
            best_conf = conf
            best_px, best_py, best_pw, best_ph = px, py, pw, ph
            best_a = jnp.zeros((_BBLK, _HW), jnp.int32)
        else:
            upd = iou > best_iou
            best_iou = jnp.where(upd, iou, best_iou)
            best_conf = jnp.where(upd, conf, best_conf)
            best_px = jnp.where(upd, px, best_px)
            best_py = jnp.where(upd, py, best_py)
            best_pw = jnp.where(upd, pw, best_pw)
            best_ph = jnp.where(upd, ph, best_ph)
            best_a = jnp.where(upd, a, best_a)

    best_cls = pred_ref[0:_NC]
    for a in range(1, _NA):
        upd3 = (best_a == a)[None]
        best_cls = jnp.where(upd3, pred_ref[a * 25:a * 25 + _NC], best_cls)

    obj = (gt_conf > 0.0).astype(jnp.float32)
    noobj = (gt_conf < 1.0).astype(jnp.float32)

    box_sq = ((best_px - gt_x) ** 2 + (best_py - gt_y) ** 2 +
              (best_pw - gt_w) ** 2 + (best_ph - gt_h) ** 2)
    box_sum = jnp.sum(obj * box_sq)

    conf_sq = (best_conf - gt_conf) ** 2
    conf_sum = jnp.sum(obj * conf_sq)
    noobj_sum = jnp.sum(noobj * conf_sq)

    lbl_val = tgt_ref[0]
    lbl = jnp.zeros((_BBLK, _HW), jnp.int32)
    for c in range(1, _NC):
        v = tgt_ref[c]
        upd = v > lbl_val
        lbl_val = jnp.where(upd, v, lbl_val)
        lbl = jnp.where(upd, c, lbl)

    m = jnp.max(best_cls, axis=0)
    lse = jnp.log(jnp.sum(jnp.exp(best_cls - m[None]), axis=0)) + m
    picked = best_cls[0]
    for c in range(1, _NC):
        picked = jnp.where(lbl == c, best_cls[c], picked)
    logp_picked = picked - lse

    cls_num = jnp.sum(obj * logp_picked)
    obj_cnt = jnp.sum(obj)

    i = pl.program_id(0)

    @pl.when(i == 0)
    def _init():
        acc_ref[0] = box_sum
        acc_ref[1] = conf_sum
        acc_ref[2] = noobj_sum
        acc_ref[3] = cls_num
        acc_ref[4] = obj_cnt

    @pl.when(i > 0)
    def _acc():
        acc_ref[0] += box_sum
        acc_ref[1] += conf_sum
        acc_ref[2] += noobj_sum
        acc_ref[3] += cls_num
        acc_ref[4] += obj_cnt

    @pl.when(i == _NSTEP - 1)
    def _fin():
        o0_ref[0] = (1.0 / _B) * _LAMBDA_COORD * acc_ref[0]
        o1_ref[0] = (1.0 / _B) * acc_ref[1]
        o2_ref[0] = (1.0 / _B) * _LAMBDA_NOOBJ * acc_ref[2]
        o3_ref[0] = -acc_ref[3] / acc_ref[4]


def kernel(prediction, target):
    pred = jnp.transpose(prediction.reshape(_B, 125, _HW), (1, 0, 2))
    tgt = jnp.transpose(target.reshape(_B, _HW, 25), (2, 0, 1))
    sd = jax.ShapeDtypeStruct((1,), jnp.float32)
    sspec = pl.BlockSpec(memory_space=pltpu.SMEM)
    o0, o1, o2, o3 = pl.pallas_call(
        _loss_kernel,
        grid=(_NSTEP,),
        out_shape=(sd, sd, sd, sd),
        in_specs=[
            pl.BlockSpec((125, _BBLK, _HW), lambda i: (0, i, 0)),
            pl.BlockSpec((25, _BBLK, _HW), lambda i: (0, i, 0)),
        ],
        out_specs=(sspec, sspec, sspec, sspec),
        scratch_shapes=[pltpu.SMEM((8,), jnp.float32)],
        compiler_params=pltpu.CompilerParams(
            dimension_semantics=("arbitrary",)),
    )(pred, tgt)
    return (o0[0], o1[0], o2[0], o3[0])


# speedup vs baseline: 1.0486x; 1.0486x over previous
import jax
import jax.numpy as jnp
from jax.experimental import pallas as pl
from jax.experimental.pallas import tpu as pltpu

_B = 64
_HW = 169
_NC = 20
_NA = 5
_AW = (1.3221, 3.19275, 5.05587, 9.47112, 11.2364)
_AH = (1.73145, 4.00944, 8.09892, 4.84053, 10.0071)
_LAMBDA_COORD = 5.0
_LAMBDA_NOOBJ = 0.5


_BBLK = 32
_NSTEP = _B // _BBLK


def _loss_kernel(pred_ref, tgt_ref, o0_ref, o1_ref, o2_ref, o3_ref, acc_ref):
    idx = jax.lax.broadcasted_iota(jnp.int32, (_BBLK, _HW), 1)
    gx = (idx % 13).astype(jnp.float32)
    gy = (idx // 13).astype(jnp.float32)

    gt_conf = tgt_ref[20]
    gt_x = tgt_ref[21]
    gt_y = tgt_ref[22]
    gt_w = tgt_ref[23]
    gt_h = tgt_ref[24]
    b_l = gt_x - gt_w * 0.5
    b_t = gt_y - gt_h * 0.5
    b_r = gt_x + gt_w * 0.5
    b_b = gt_y + gt_h * 0.5
    area_b = gt_w * gt_h

    best_iou = None
    for a in range(_NA):
        base = a * 25
        conf = jax.nn.sigmoid(pred_ref[base + 20])
        px = jax.nn.sigmoid(pred_ref[base + 21]) + gx
        py = jax.nn.sigmoid(pred_ref[base + 22]) + gy
        pw = jnp.exp(pred_ref[base + 23]) * _AW[a]
        ph = jnp.exp(pred_ref[base + 24]) * _AH[a]

        a_l = px - pw * 0.5
        a_t = py - ph * 0.5
        a_r = px + pw * 0.5
        a_b = py + ph * 0.5
        wi = jnp.clip(jnp.minimum(a_r, b_r) - jnp.maximum(a_l, b_l), 0.0)
        hi = jnp.clip(jnp.minimum(a_b, b_b) - jnp.maximum(a_t, b_t), 0.0)
        inter = wi * hi
        iou = inter / (pw * ph + area_b - inter + 1e-8)

        if a == 0:
            best_iou = iou
            best_conf = conf
            best_px, best_py, best_pw, best_ph = px, py, pw, ph
            best_a = jnp.zeros((_BBLK, _HW), jnp.int32)
        else:
            upd = iou > best_iou
            best_iou = jnp.where(upd, iou, best_iou)
            best_conf = jnp.where(upd, conf, best_conf)
            best_px = jnp.where(upd, px, best_px)
            best_py = jnp.where(upd, py, best_py)
            best_pw = jnp.where(upd, pw, best_pw)
            best_ph = jnp.where(upd, ph, best_ph)
            best_a = jnp.where(upd, a, best_a)

    best_cls = pred_ref[0:_NC]
    for a in range(1, _NA):
        upd3 = (best_a == a)[None]
        best_cls = jnp.where(upd3, pred_ref[a * 25:a * 25 + _NC], best_cls)

    obj = (gt_conf > 0.0).astype(jnp.float32)
    noobj = (gt_conf < 1.0).astype(jnp.float32)

    box_sq = ((best_px - gt_x) ** 2 + (best_py - gt_y) ** 2 +
              (best_pw - gt_w) ** 2 + (best_ph - gt_h) ** 2)
    box_sum = jnp.sum(obj * box_sq)

    conf_sq = (best_conf - gt_conf) ** 2
    conf_sum = jnp.sum(obj * conf_sq)
    noobj_sum = jnp.sum(noobj * conf_sq)

    lbl_val = tgt_ref[0]
    lbl = jnp.zeros((_BBLK, _HW), jnp.int32)
    for c in range(1, _NC):
        v = tgt_ref[c]
        upd = v > lbl_val
        lbl_val = jnp.where(upd, v, lbl_val)
        lbl = jnp.where(upd, c, lbl)

    m = jnp.max(best_cls, axis=0)
    lse = jnp.log(jnp.sum(jnp.exp(best_cls - m[None]), axis=0)) + m
    picked = best_cls[0]
    for c in range(1, _NC):
        picked = jnp.where(lbl == c, best_cls[c], picked)
    logp_picked = picked - lse

    cls_num = jnp.sum(obj * logp_picked)
    obj_cnt = jnp.sum(obj)

    i = pl.program_id(0)

    @pl.when(i == 0)
    def _init():
        acc_ref[0] = box_sum
        acc_ref[1] = conf_sum
        acc_ref[2] = noobj_sum
        acc_ref[3] = cls_num
        acc_ref[4] = obj_cnt

    @pl.when(i > 0)
    def _acc():
        acc_ref[0] += box_sum
        acc_ref[1] += conf_sum
        acc_ref[2] += noobj_sum
        acc_ref[3] += cls_num
        acc_ref[4] += obj_cnt

    @pl.when(i == _NSTEP - 1)
    def _fin():
        o0_ref[0] = (1.0 / _B) * _LAMBDA_COORD * acc_ref[0]
        o1_ref[0] = (1.0 / _B) * acc_ref[1]
        o2_ref[0] = (1.0 / _B) * _LAMBDA_NOOBJ * acc_ref[2]
        o3_ref[0] = -acc_ref[3] / acc_ref[4]


def kernel(prediction, target):
    pred = jnp.transpose(prediction.reshape(_B, 125, _HW), (1, 0, 2))
    tgt = jnp.transpose(target.reshape(_B, _HW, 25), (2, 0, 1))
    sd = jax.ShapeDtypeStruct((1,), jnp.float32)
    sspec = pl.BlockSpec(memory_space=pltpu.SMEM)
    o0, o1, o2, o3 = pl.pallas_call(
        _loss_kernel,
        grid=(_NSTEP,),
        out_shape=(sd, sd, sd, sd),
        in_specs=[
            pl.BlockSpec((125, _BBLK, _HW), lambda i: (0, i, 0)),
            pl.BlockSpec((25, _BBLK, _HW), lambda i: (0, i, 0)),
        ],
        out_specs=(sspec, sspec, sspec, sspec),
        scratch_shapes=[pltpu.SMEM((8,), jnp.float32)],
        compiler_params=pltpu.CompilerParams(
            dimension_semantics=("arbitrary",)),
    )(pred, tgt)
    return (o0[0], o1[0], o2[0], o3[0])
